# baseline (speedup 1.0000x reference)
import functools

import jax
import jax.numpy as jnp
from jax import lax
from jax.experimental import pallas as pl
from jax.experimental.pallas import tpu as pltpu
from jax.experimental.pallas import tpu_sc as plsc

_N = 10000
_E = 320000
_ROW_BLK = 400
_NW = 32
_EPW = 10240
_EP = _EPW * _NW
_CB = 128
_NCH = _EPW // _CB
_NP = 10240


def _matmul_body(x_ref, w_ref, o_ref):
    o_ref[...] = jnp.dot(x_ref[...], w_ref[...],
                         preferred_element_type=jnp.float32)


def _matmul(x, w):
    n, k = x.shape
    m = w.shape[1]
    return pl.pallas_call(
        _matmul_body,
        grid=(n // _ROW_BLK,),
        in_specs=[
            pl.BlockSpec((_ROW_BLK, k), lambda i: (i, 0)),
            pl.BlockSpec((k, m), lambda i: (0, 0)),
        ],
        out_specs=pl.BlockSpec((_ROW_BLK, m), lambda i: (i, 0)),
        out_shape=jax.ShapeDtypeStruct((n, m), jnp.float32),
    )(x, w)


@functools.lru_cache(maxsize=None)
def _make_att():
    mesh = plsc.VectorSubcoreMesh(core_axis_name="c", subcore_axis_name="s")

    @functools.partial(
        pl.kernel,
        mesh=mesh,
        out_type=jax.ShapeDtypeStruct((_EP, 128), jnp.float32),
        scratch_types=[
            pltpu.VMEM((_CB,), jnp.int32),
            pltpu.VMEM((_CB,), jnp.int32),
            pltpu.VMEM((_CB, 128), jnp.float32),
            pltpu.VMEM((_CB, 128), jnp.float32),
            pltpu.VMEM((16,), jnp.float32),
            pltpu.SemaphoreType.DMA,
            pltpu.SemaphoreType.DMA,
        ],
    )
    def att(att128, srcv, dstv, crep, ex_out,
            src_v, dst_v, as_v, ad_v, c_v, sem, sem2):
        cid = lax.axis_index("c")
        sid = lax.axis_index("s")
        wid = sid * 2 + cid
        base = wid * _EPW
        pltpu.sync_copy(crep, c_v)

        def cbody(k, carry):
            e0 = base + k * _CB
            pltpu.sync_copy(srcv.at[pl.ds(e0, _CB)], src_v)
            pltpu.sync_copy(dstv.at[pl.ds(e0, _CB)], dst_v)
            cp1 = pltpu.async_copy(att128.at[src_v], as_v, sem)
            cp2 = pltpu.async_copy(att128.at[dst_v], ad_v, sem2)
            cp1.wait()
            cp2.wait()

            def sbody(i, c2):
                s = as_v[i, pl.ds(0, 16)] + ad_v[i, pl.ds(16, 16)]
                s = jnp.where(s > 0, s, s * jnp.float32(0.2))
                ex = jnp.exp(s - c_v[:])
                for jj in range(8):
                    as_v[i, pl.ds(jj * 16, 16)] = ex
                return c2
            lax.fori_loop(0, _CB, sbody, 0)
            pltpu.sync_copy(as_v, ex_out.at[pl.ds(e0, _CB)])
            return carry
        lax.fori_loop(0, _NCH, cbody, 0)

    return att


@functools.lru_cache(maxsize=None)
def _make_agg(P, H):
    S = P // H
    mesh = plsc.VectorSubcoreMesh(core_axis_name="c", subcore_axis_name="s")

    @functools.partial(
        pl.kernel,
        mesh=mesh,
        out_type=jax.ShapeDtypeStruct((2, P + 1, _NP, 128), jnp.float32),
        scratch_types=[
            pltpu.VMEM((_CB,), jnp.int32),
            pltpu.VMEM((_CB,), jnp.int32),
            pltpu.VMEM((_CB,), jnp.int32),
            pltpu.VMEM((_CB,), jnp.int32),
            pltpu.VMEM((_CB * 16,), jnp.float32),
            pltpu.VMEM((_CB * 16,), jnp.float32),
            pltpu.VMEM((_CB, 128), jnp.float32),
            pltpu.VMEM((_CB, 128), jnp.float32),
            pltpu.SemaphoreType.DMA,
            pltpu.SemaphoreType.DMA,
            pltpu.VMEM_SHARED((_NP, 128), jnp.float32),
        ],
    )
    def agg(zrows, idxp, dstv, w, exrows, zeros_hbm, out,
            idx_v0, idx_v1, dst_v0, dst_v1, w_v0, w_v1, rows_v0, rows_v1,
            sem_g0, sem_g1, acc):
        cid = lax.axis_index("c")
        sid = lax.axis_index("s")
        wid = sid * 2 + cid
        base = wid * _EPW
        idx_v = (idx_v0, idx_v1)
        dst_v = (dst_v0, dst_v1)
        w_v = (w_v0, w_v1)
        rows_v = (rows_v0, rows_v1)
        sem_g = (sem_g0, sem_g1)

        def init_acc():
            pltpu.sync_copy(zeros_hbm, acc.at[pl.ds(sid * 640, 640)])

        def smalls(p, h, c, b):
            e0 = base + c * _CB
            pltpu.sync_copy(idxp.at[pl.ds(p * _EP + e0, _CB)], idx_v[b])
            pltpu.sync_copy(dstv.at[pl.ds(e0, _CB)], dst_v[b])
            pltpu.sync_copy(w.at[pl.ds((h * _EP + e0) * 16, _CB * 16)],
                            w_v[b])

        def scale(b):
            def sb(i, c2):
                for u in range(2):
                    e = 2 * i + u
                    wv = w_v[b][pl.ds(e * 16, 16)]
                    for jj in range(8):
                        sl = pl.ds(jj * 16, 16)
                        rows_v[b][e, sl] = rows_v[b][e, sl] * wv
                return c2
            lax.fori_loop(0, _CB // 2, sb, 0)

        for p in range(P):
            h = p // S
            init_acc()
            plsc.subcore_barrier()
            smalls(p, h, 0, 0)
            pltpu.async_copy(zrows.at[idx_v[0]], rows_v[0], sem_g[0])

            def body(j, carry):
                for b in range(2):
                    c = 2 * j + b
                    pltpu.make_async_copy(zrows.at[idx_v[b]], rows_v[b],
                                          sem_g[b]).wait()

                    @pl.when(c + 1 < _NCH)
                    def _prefetch():
                        smalls(p, h, c + 1, 1 - b)
                        pltpu.async_copy(zrows.at[idx_v[1 - b]],
                                         rows_v[1 - b], sem_g[1 - b])
                    scale(b)
                    pltpu.sync_copy(rows_v[b], acc.at[dst_v[b]], add=True)
                return carry
            lax.fori_loop(0, _NCH // 2, body, 0)
            plsc.subcore_barrier()
            pltpu.sync_copy(acc.at[pl.ds(sid * 640, 640)],
                            out.at[cid, p, pl.ds(sid * 640, 640)])
            plsc.subcore_barrier()

        init_acc()
        plsc.subcore_barrier()

        def dbody(k, carry):
            e0 = base + k * _CB
            pltpu.sync_copy(dstv.at[pl.ds(e0, _CB)], dst_v0)
            pltpu.sync_copy(exrows.at[pl.ds(e0, _CB)], rows_v0)
            pltpu.sync_copy(rows_v0, acc.at[dst_v0], add=True)
            return carry
        lax.fori_loop(0, _NCH, dbody, 0)
        plsc.subcore_barrier()
        pltpu.sync_copy(acc.at[pl.ds(sid * 640, 640)],
                        out.at[cid, P, pl.ds(sid * 640, 640)])

    return agg


def _gat(x, src, dst, idxp, W, a_src, a_dst, heads, out_dim):
    P = heads * out_dim // 128
    S = P // heads
    z = _matmul(x, W)

    eye = jnp.eye(16, dtype=jnp.float32)[:heads]
    ar = a_src.reshape(heads, S, 128)
    ad = a_dst.reshape(heads, S, 128)
    A1 = ar[:, :, :, None] * eye[:, None, None, :]
    A2 = ad[:, :, :, None] * eye[:, None, None, :]
    Amat = jnp.concatenate(
        [A1, A2, jnp.zeros((heads, S, 128, 96), jnp.float32)],
        axis=-1).reshape(P * 128, 128)
    att128 = _matmul(z, Amat)
    lanes = jnp.arange(16) < heads
    c16 = jnp.where(lanes,
                    att128[:, :16].max(axis=0) + att128[:, 16:32].max(axis=0),
                    jnp.float32(100.0))

    ex128 = _make_att()(att128, src, dst, c16)
    ex16 = ex128[:, :16]
    w16 = jnp.broadcast_to(ex16[:, :heads].T.reshape(-1)[:, None],
                           (heads * _EP, 16)).reshape(-1)
    zeros_hbm = jnp.zeros((640, 128), jnp.float32)
    raw = _make_agg(P, heads)(z.reshape(_N * P, 128), idxp.reshape(-1), dst,
                              w16, ex128, zeros_hbm)
    den = (raw[0, P] + raw[1, P])[:_N, :heads] + jnp.float32(1e-30)
    out = (raw[0, :P] + raw[1, :P])[:, :_N, :]
    out = out / jnp.repeat(den.T, S, axis=0)[:, :, None]
    out = out.reshape(heads, S, _N, 128).transpose(2, 0, 1, 3)
    return out.reshape(_N, heads, S * 128)


def kernel(input_matrix, adjs, W1, a1_src, a1_dst, W2, a2_src, a2_dst,
           W3, a3_src, a3_dst):
    src = adjs[0].astype(jnp.int32)
    dst = adjs[1].astype(jnp.int32)
    npad = _EP - _E
    src_p = jnp.concatenate([src, jnp.zeros((npad,), jnp.int32)])
    dst_p = jnp.concatenate([dst, jnp.full((npad,), _NP - 1, jnp.int32)])
    idx8 = src_p[None, :] * 8 + jnp.arange(8, dtype=jnp.int32)[:, None]
    idx6 = src_p[None, :] * 6 + jnp.arange(6, dtype=jnp.int32)[:, None]

    h1 = jax.nn.elu(
        _gat(input_matrix, src_p, dst_p, idx8, W1, a1_src, a1_dst, 4, 256)
        .reshape(_N, 1024))
    h2 = jax.nn.elu(
        _gat(h1, src_p, dst_p, idx8, W2, a2_src, a2_dst, 4, 256)
        .reshape(_N, 1024)) + h1

    W3p = jnp.pad(W3.reshape(1024, 6, 121), ((0, 0), (0, 0), (0, 7)))
    W3p = W3p.reshape(1024, 6 * 128)
    a3s = jnp.pad(a3_src, ((0, 0), (0, 7)))
    a3d = jnp.pad(a3_dst, ((0, 0), (0, 7)))
    out3 = _gat(h2, src_p, dst_p, idx6, W3p, a3s, a3d, 6, 128)
    h3 = out3.mean(axis=1)[:, :121]
    return jax.nn.log_softmax(h3, axis=1)


# speedup vs baseline: 1.2559x; 1.2559x over previous
import functools

import jax
import jax.numpy as jnp
from jax import lax
from jax.experimental import pallas as pl
from jax.experimental.pallas import tpu as pltpu
from jax.experimental.pallas import tpu_sc as plsc

_N = 10000
_E = 320000
_ROW_BLK = 400
_NW = 32
_EPW = _E // _NW
_C = 80
_NCHUNK = _EPW // _C
_NP = 10240


def _matmul_body(x_ref, w_ref, o_ref):
    o_ref[...] = jnp.dot(x_ref[...], w_ref[...],
                         preferred_element_type=jnp.float32)


def _matmul(x, w):
    n, k = x.shape
    m = w.shape[1]
    return pl.pallas_call(
        _matmul_body,
        grid=(n // _ROW_BLK,),
        in_specs=[
            pl.BlockSpec((_ROW_BLK, k), lambda i: (i, 0)),
            pl.BlockSpec((k, m), lambda i: (0, 0)),
        ],
        out_specs=pl.BlockSpec((_ROW_BLK, m), lambda i: (i, 0)),
        out_shape=jax.ShapeDtypeStruct((n, m), jnp.float32),
    )(x, w)


@functools.lru_cache(maxsize=None)
def _make_agg(P, H):
    S = P // H
    mesh = plsc.VectorSubcoreMesh(core_axis_name="c", subcore_axis_name="s")

    @functools.partial(
        pl.kernel,
        mesh=mesh,
        out_type=jax.ShapeDtypeStruct((2, P, _NP, 128), jnp.float32),
        scratch_types=[
            pltpu.VMEM((_C,), jnp.int32),
            pltpu.VMEM((_C,), jnp.int32),
            pltpu.VMEM((_C * 16,), jnp.float32),
            pltpu.VMEM((_C, 128), jnp.float32),
            pltpu.VMEM((128, 128), jnp.float32),
            pltpu.SemaphoreType.DMA,
            pltpu.VMEM_SHARED((_NP, 128), jnp.float32),
        ],
    )
    def agg(zrows, idxp, dstv, w, out, idx_v, dst_v, w_v, rows_v, zeros_v,
            sem, acc):
        cid = lax.axis_index("c")
        sid = lax.axis_index("s")
        wid = sid * 2 + cid
        base = wid * _EPW

        def zbody(j, carry):
            for jj in range(8):
                zeros_v[j, pl.ds(jj * 16, 16)] = jnp.zeros((16,), jnp.float32)
            return carry
        lax.fori_loop(0, 128, zbody, 0)

        for p in range(P):
            h = p // S
            for t in range(5):
                pltpu.sync_copy(zeros_v,
                                acc.at[pl.ds(sid * 640 + t * 128, 128)])
            plsc.subcore_barrier()

            def cbody(k, carry):
                e0 = base + k * _C
                pltpu.sync_copy(idxp.at[pl.ds(p * _E + e0, _C)], idx_v)
                pltpu.sync_copy(dstv.at[pl.ds(e0, _C)], dst_v)
                pltpu.sync_copy(w.at[pl.ds((h * _E + e0) * 16, _C * 16)],
                                w_v)
                pltpu.async_copy(zrows.at[idx_v], rows_v, sem).wait()

                def sbody(i, c2):
                    wv = w_v[pl.ds(i * 16, 16)]
                    for jj in range(8):
                        sl = pl.ds(jj * 16, 16)
                        rows_v[i, sl] = rows_v[i, sl] * wv
                    return c2
                lax.fori_loop(0, _C, sbody, 0)
                pltpu.sync_copy(rows_v, acc.at[dst_v], add=True)
                return carry
            lax.fori_loop(0, _NCHUNK, cbody, 0)
            plsc.subcore_barrier()
            pltpu.sync_copy(acc.at[pl.ds(sid * 640, 640)],
                            out.at[cid, p, pl.ds(sid * 640, 640)])
            plsc.subcore_barrier()

    return agg


@functools.lru_cache(maxsize=None)
def _make_att():
    mesh = plsc.VectorSubcoreMesh(core_axis_name="c", subcore_axis_name="s")

    @functools.partial(
        pl.kernel,
        mesh=mesh,
        out_type=jax.ShapeDtypeStruct((_E, 128), jnp.float32),
        scratch_types=[
            pltpu.VMEM((_C,), jnp.int32),
            pltpu.VMEM((_C,), jnp.int32),
            pltpu.VMEM((_C, 128), jnp.float32),
            pltpu.VMEM((_C, 128), jnp.float32),
            pltpu.VMEM((16,), jnp.float32),
            pltpu.SemaphoreType.DMA,
            pltpu.SemaphoreType.DMA,
        ],
    )
    def att(att128, srcv, dstv, crep, ex_out,
            src_v, dst_v, as_v, ad_v, c_v, sem, sem2):
        cid = lax.axis_index("c")
        sid = lax.axis_index("s")
        wid = sid * 2 + cid
        base = wid * _EPW
        pltpu.sync_copy(crep, c_v)

        def cbody(k, carry):
            e0 = base + k * _C
            pltpu.sync_copy(srcv.at[pl.ds(e0, _C)], src_v)
            pltpu.sync_copy(dstv.at[pl.ds(e0, _C)], dst_v)
            cp1 = pltpu.async_copy(att128.at[src_v], as_v, sem)
            cp2 = pltpu.async_copy(att128.at[dst_v], ad_v, sem2)
            cp1.wait()
            cp2.wait()

            def sbody(i, c2):
                s = as_v[i, pl.ds(0, 16)] + ad_v[i, pl.ds(16, 16)]
                s = jnp.where(s > 0, s, s * jnp.float32(0.2))
                ex = jnp.exp(s - c_v[:])
                for jj in range(8):
                    as_v[i, pl.ds(jj * 16, 16)] = ex
                return c2
            lax.fori_loop(0, _C, sbody, 0)
            pltpu.sync_copy(as_v, ex_out.at[pl.ds(e0, _C)])
            return carry
        lax.fori_loop(0, _NCHUNK, cbody, 0)

    return att


def _gat(x, src, dst, idxp, W, a_src, a_dst, heads, out_dim):
    P = heads * out_dim // 128
    S = P // heads
    z = _matmul(x, W)

    eye = jnp.eye(16, dtype=jnp.float32)[:heads]
    ar = a_src.reshape(heads, S, 128)
    ad = a_dst.reshape(heads, S, 128)
    A1 = ar[:, :, :, None] * eye[:, None, None, :]
    A2 = ad[:, :, :, None] * eye[:, None, None, :]
    Amat = jnp.concatenate(
        [A1, A2, jnp.zeros((heads, S, 128, 96), jnp.float32)],
        axis=-1).reshape(P * 128, 128)
    att128 = _matmul(z, Amat)
    lanes = jnp.arange(16) < heads
    c16 = jnp.where(lanes,
                    att128[:, :16].max(axis=0) + att128[:, 16:32].max(axis=0),
                    jnp.float32(100.0))

    ex128 = _make_att()(att128, src, dst, c16)
    ex16 = ex128[:, :16]
    w16 = jnp.broadcast_to(ex16[:, :heads].T.reshape(-1)[:, None],
                           (heads * _E, 16)).reshape(-1)
    raw = _make_agg(P, heads)(z.reshape(_N * P, 128), idxp.reshape(-1), dst,
                              w16)
    den16 = jax.ops.segment_sum(ex16[:, :heads], dst, num_segments=_N)
    den = den16 + jnp.float32(1e-30)
    out = (raw[0] + raw[1])[:, :_N, :]
    out = out / jnp.repeat(den.T, S, axis=0)[:, :, None]
    out = out.reshape(heads, S, _N, 128).transpose(2, 0, 1, 3)
    return out.reshape(_N, heads, S * 128)


def kernel(input_matrix, adjs, W1, a1_src, a1_dst, W2, a2_src, a2_dst,
           W3, a3_src, a3_dst):
    src = adjs[0].astype(jnp.int32)
    dst = adjs[1].astype(jnp.int32)
    idx8 = src[None, :] * 8 + jnp.arange(8, dtype=jnp.int32)[:, None]
    idx6 = src[None, :] * 6 + jnp.arange(6, dtype=jnp.int32)[:, None]

    h1 = jax.nn.elu(
        _gat(input_matrix, src, dst, idx8, W1, a1_src, a1_dst, 4, 256)
        .reshape(_N, 1024))
    h2 = jax.nn.elu(
        _gat(h1, src, dst, idx8, W2, a2_src, a2_dst, 4, 256)
        .reshape(_N, 1024)) + h1

    W3p = jnp.pad(W3.reshape(1024, 6, 121), ((0, 0), (0, 0), (0, 7)))
    W3p = W3p.reshape(1024, 6 * 128)
    a3s = jnp.pad(a3_src, ((0, 0), (0, 7)))
    a3d = jnp.pad(a3_dst, ((0, 0), (0, 7)))
    out3 = _gat(h2, src, dst, idx6, W3p, a3s, a3d, 6, 128)
    h3 = out3.mean(axis=1)[:, :121]
    return jax.nn.log_softmax(h3, axis=1)


# speedup vs baseline: 1.5158x; 1.2069x over previous
import functools

import jax
import jax.numpy as jnp
from jax import lax
from jax.experimental import pallas as pl
from jax.experimental.pallas import tpu as pltpu
from jax.experimental.pallas import tpu_sc as plsc

_N = 10000
_E = 320000
_ROW_BLK = 400
_NW = 32
_EPW = _E // _NW
_C = 80
_NCHUNK = _EPW // _C
_NP = 10240


def _matmul_body(x_ref, w_ref, o_ref):
    o_ref[...] = jnp.dot(x_ref[...], w_ref[...],
                         preferred_element_type=jnp.float32)


def _matmul(x, w):
    n, k = x.shape
    m = w.shape[1]
    return pl.pallas_call(
        _matmul_body,
        grid=(n // _ROW_BLK,),
        in_specs=[
            pl.BlockSpec((_ROW_BLK, k), lambda i: (i, 0)),
            pl.BlockSpec((k, m), lambda i: (0, 0)),
        ],
        out_specs=pl.BlockSpec((_ROW_BLK, m), lambda i: (i, 0)),
        out_shape=jax.ShapeDtypeStruct((n, m), jnp.float32),
    )(x, w)


@functools.lru_cache(maxsize=None)
def _make_agg(P, H):
    S = P // H
    mesh = plsc.VectorSubcoreMesh(core_axis_name="c", subcore_axis_name="s")

    @functools.partial(
        pl.kernel,
        mesh=mesh,
        out_type=jax.ShapeDtypeStruct((2, P, _NP, 128), jnp.float32),
        scratch_types=[
            pltpu.VMEM((_C,), jnp.int32),
            pltpu.VMEM((_C,), jnp.int32),
            pltpu.VMEM((_C,), jnp.int32),
            pltpu.VMEM((_C,), jnp.int32),
            pltpu.VMEM((_C * 16,), jnp.float32),
            pltpu.VMEM((_C * 16,), jnp.float32),
            pltpu.VMEM((_C, 128), jnp.float32),
            pltpu.VMEM((_C, 128), jnp.float32),
            pltpu.VMEM((128, 128), jnp.float32),
            pltpu.SemaphoreType.DMA,
            pltpu.SemaphoreType.DMA,
            pltpu.VMEM_SHARED((_NP, 128), jnp.float32),
        ],
    )
    def agg(zrows, idxp, dstv, w, out, idx_v0, idx_v1, dst_v0, dst_v1,
            w_v0, w_v1, rows_v0, rows_v1, zeros_v, sem_g0, sem_g1, acc):
        cid = lax.axis_index("c")
        sid = lax.axis_index("s")
        wid = sid * 2 + cid
        base = wid * _EPW
        idx_v = (idx_v0, idx_v1)
        dst_v = (dst_v0, dst_v1)
        w_v = (w_v0, w_v1)
        rows_v = (rows_v0, rows_v1)
        sem_g = (sem_g0, sem_g1)

        def zbody(j, carry):
            for jj in range(8):
                zeros_v[j, pl.ds(jj * 16, 16)] = jnp.zeros((16,), jnp.float32)
            return carry
        lax.fori_loop(0, 128, zbody, 0)

        def smalls(p, h, c, b):
            e0 = base + c * _C
            pltpu.sync_copy(idxp.at[pl.ds(p * _E + e0, _C)], idx_v[b])
            pltpu.sync_copy(dstv.at[pl.ds(e0, _C)], dst_v[b])
            pltpu.sync_copy(w.at[pl.ds((h * _E + e0) * 16, _C * 16)],
                            w_v[b])

        def process(b):
            pltpu.make_async_copy(zrows.at[idx_v[b]], rows_v[b],
                                  sem_g[b]).wait()

            def sbody(i, c2):
                wv = w_v[b][pl.ds(i * 16, 16)]
                for jj in range(8):
                    sl = pl.ds(jj * 16, 16)
                    rows_v[b][i, sl] = rows_v[b][i, sl] * wv
                return c2
            lax.fori_loop(0, _C, sbody, 0)
            pltpu.sync_copy(rows_v[b], acc.at[dst_v[b]], add=True)

        for p in range(P):
            h = p // S
            for t in range(5):
                pltpu.sync_copy(zeros_v,
                                acc.at[pl.ds(sid * 640 + t * 128, 128)])
            plsc.subcore_barrier()
            smalls(p, h, 0, 0)
            pltpu.async_copy(zrows.at[idx_v[0]], rows_v[0], sem_g[0])

            def body(j, carry):
                smalls(p, h, 2 * j + 1, 1)
                pltpu.async_copy(zrows.at[idx_v[1]], rows_v[1], sem_g[1])
                process(0)
                smalls(p, h, 2 * j + 2, 0)
                pltpu.async_copy(zrows.at[idx_v[0]], rows_v[0], sem_g[0])
                process(1)
                return carry
            lax.fori_loop(0, _NCHUNK // 2, body, 0)
            process(0)
            plsc.subcore_barrier()
            pltpu.sync_copy(acc.at[pl.ds(sid * 640, 640)],
                            out.at[cid, p, pl.ds(sid * 640, 640)])
            plsc.subcore_barrier()

    return agg


@functools.lru_cache(maxsize=None)
def _make_att():
    mesh = plsc.VectorSubcoreMesh(core_axis_name="c", subcore_axis_name="s")

    @functools.partial(
        pl.kernel,
        mesh=mesh,
        out_type=jax.ShapeDtypeStruct((_E, 128), jnp.float32),
        scratch_types=[
            pltpu.VMEM((_C,), jnp.int32),
            pltpu.VMEM((_C,), jnp.int32),
            pltpu.VMEM((_C, 128), jnp.float32),
            pltpu.VMEM((_C, 128), jnp.float32),
            pltpu.VMEM((16,), jnp.float32),
            pltpu.SemaphoreType.DMA,
            pltpu.SemaphoreType.DMA,
        ],
    )
    def att(att128, srcv, dstv, crep, ex_out,
            src_v, dst_v, as_v, ad_v, c_v, sem, sem2):
        cid = lax.axis_index("c")
        sid = lax.axis_index("s")
        wid = sid * 2 + cid
        base = wid * _EPW
        pltpu.sync_copy(crep, c_v)

        def cbody(k, carry):
            e0 = base + k * _C
            pltpu.sync_copy(srcv.at[pl.ds(e0, _C)], src_v)
            pltpu.sync_copy(dstv.at[pl.ds(e0, _C)], dst_v)
            cp1 = pltpu.async_copy(att128.at[src_v], as_v, sem)
            cp2 = pltpu.async_copy(att128.at[dst_v], ad_v, sem2)
            cp1.wait()
            cp2.wait()

            def sbody(i, c2):
                s = as_v[i, pl.ds(0, 16)] + ad_v[i, pl.ds(16, 16)]
                s = jnp.where(s > 0, s, s * jnp.float32(0.2))
                ex = jnp.exp(s - c_v[:])
                for jj in range(8):
                    as_v[i, pl.ds(jj * 16, 16)] = ex
                return c2
            lax.fori_loop(0, _C, sbody, 0)
            pltpu.sync_copy(as_v, ex_out.at[pl.ds(e0, _C)])
            return carry
        lax.fori_loop(0, _NCHUNK, cbody, 0)

    return att


def _gat(x, src, dst, idxp, W, a_src, a_dst, heads, out_dim):
    P = heads * out_dim // 128
    S = P // heads
    z = _matmul(x, W)

    eye = jnp.eye(16, dtype=jnp.float32)[:heads]
    ar = a_src.reshape(heads, S, 128)
    ad = a_dst.reshape(heads, S, 128)
    A1 = ar[:, :, :, None] * eye[:, None, None, :]
    A2 = ad[:, :, :, None] * eye[:, None, None, :]
    Amat = jnp.concatenate(
        [A1, A2, jnp.zeros((heads, S, 128, 96), jnp.float32)],
        axis=-1).reshape(P * 128, 128)
    att128 = _matmul(z, Amat)
    lanes = jnp.arange(16) < heads
    c16 = jnp.where(lanes,
                    att128[:, :16].max(axis=0) + att128[:, 16:32].max(axis=0),
                    jnp.float32(100.0))

    ex128 = _make_att()(att128, src, dst, c16)
    ex16 = ex128[:, :16]
    w16 = jnp.broadcast_to(ex16[:, :heads].T.reshape(-1)[:, None],
                           (heads * _E, 16)).reshape(-1)
    raw = _make_agg(P, heads)(z.reshape(_N * P, 128), idxp.reshape(-1), dst,
                              w16)
    den16 = jax.ops.segment_sum(ex16[:, :heads], dst, num_segments=_N)
    den = den16 + jnp.float32(1e-30)
    out = (raw[0] + raw[1])[:, :_N, :]
    out = out / jnp.repeat(den.T, S, axis=0)[:, :, None]
    out = out.reshape(heads, S, _N, 128).transpose(2, 0, 1, 3)
    return out.reshape(_N, heads, S * 128)


def kernel(input_matrix, adjs, W1, a1_src, a1_dst, W2, a2_src, a2_dst,
           W3, a3_src, a3_dst):
    src = adjs[0].astype(jnp.int32)
    dst = adjs[1].astype(jnp.int32)
    idx8 = src[None, :] * 8 + jnp.arange(8, dtype=jnp.int32)[:, None]
    idx6 = src[None, :] * 6 + jnp.arange(6, dtype=jnp.int32)[:, None]

    h1 = jax.nn.elu(
        _gat(input_matrix, src, dst, idx8, W1, a1_src, a1_dst, 4, 256)
        .reshape(_N, 1024))
    h2 = jax.nn.elu(
        _gat(h1, src, dst, idx8, W2, a2_src, a2_dst, 4, 256)
        .reshape(_N, 1024)) + h1

    W3p = jnp.pad(W3.reshape(1024, 6, 121), ((0, 0), (0, 0), (0, 7)))
    W3p = W3p.reshape(1024, 6 * 128)
    a3s = jnp.pad(a3_src, ((0, 0), (0, 7)))
    a3d = jnp.pad(a3_dst, ((0, 0), (0, 7)))
    out3 = _gat(h2, src, dst, idx6, W3p, a3s, a3d, 6, 128)
    h3 = out3.mean(axis=1)[:, :121]
    return jax.nn.log_softmax(h3, axis=1)
